# baseline (speedup 1.0000x reference)
import jax
import jax.numpy as jnp
from jax import lax
from jax.experimental import pallas as pl
from jax.experimental.pallas import tpu as pltpu


def _embed_kernel(x_ref, na_ref, tr_ref, w_ref, b_ref, out_ref,
                  wx_ref, wna_ref, wtr_ref):
    tt = x_ref.shape[1]
    vv = x_ref.shape[2]
    na = wna_ref.shape[0]

    @pl.when(pl.program_id(0) == 0)
    def _load_w_slabs():
        for t in range(tt):
            wx_ref[t] = w_ref[t * vv:(t + 1) * vv, :]
        wna_ref[:] = w_ref[tt * vv:tt * vv + na, :]
        wtr_ref[:] = w_ref[tt * vv + na:, :]

    dn = (((0,), (0,)), ((), ()))
    acc = lax.dot_general(x_ref[0, 0, :, 0, :], wx_ref[0], dn,
                          preferred_element_type=jnp.float32)
    for t in range(1, tt):
        acc += lax.dot_general(x_ref[0, t, :, 0, :], wx_ref[t], dn,
                               preferred_element_type=jnp.float32)
    acc += lax.dot_general(na_ref[:], wna_ref[:], dn,
                           preferred_element_type=jnp.float32)
    acc += lax.dot_general(tr_ref[:], wtr_ref[:], dn,
                           preferred_element_type=jnp.float32)
    out_ref[:] = acc + b_ref[:]


def kernel(x, node_attr_data, trainable_data, node_attr_hidden,
           trainable_hidden, edge_attr, W_data, b_data, W_hidden, b_hidden,
           W_edge, Wq, Wk, Wv, Wo, W_mlp1, W_mlp2, ln1_g, ln1_b, ln2_g,
           ln2_b, edge_index):
    B, Tt, Ens, G, Vv = x.shape
    NA = node_attr_data.shape[1]
    TR = trainable_data.shape[1]
    D = W_data.shape[1]

    xt = jnp.transpose(x, (0, 1, 4, 2, 3))
    nat = node_attr_data.T
    trt = trainable_data.T
    b2 = b_data.reshape(1, D)

    Bg = 11008
    grid = (pl.cdiv(G, Bg),)

    return pl.pallas_call(
        _embed_kernel,
        grid=grid,
        in_specs=[
            pl.BlockSpec((1, Tt, Vv, 1, Bg), lambda i: (0, 0, 0, 0, i)),
            pl.BlockSpec((NA, Bg), lambda i: (0, i)),
            pl.BlockSpec((TR, Bg), lambda i: (0, i)),
            pl.BlockSpec((Tt * Vv + NA + TR, D), lambda i: (0, 0)),
            pl.BlockSpec((1, D), lambda i: (0, 0)),
        ],
        out_specs=pl.BlockSpec((Bg, D), lambda i: (i, 0)),
        out_shape=jax.ShapeDtypeStruct((G, D), jnp.float32),
        scratch_shapes=[
            pltpu.VMEM((Tt, Vv, D), jnp.float32),
            pltpu.VMEM((NA, D), jnp.float32),
            pltpu.VMEM((TR, D), jnp.float32),
        ],
    )(xt, nat, trt, W_data, b2)


# speedup vs baseline: 1.0354x; 1.0354x over previous
import jax
import jax.numpy as jnp
from jax import lax
from jax.experimental import pallas as pl
from jax.experimental.pallas import tpu as pltpu


def _embed_kernel(x_ref, na_ref, tr_ref, w_ref, b_ref, out_ref,
                  wx_ref, wna_ref, wtr_ref):
    tt = x_ref.shape[1]
    vv = x_ref.shape[2]
    na = wna_ref.shape[0]

    @pl.when(pl.program_id(0) == 0)
    def _load_w_slabs():
        for t in range(tt):
            wx_ref[t] = w_ref[t * vv:(t + 1) * vv, :]
        wna_ref[:] = w_ref[tt * vv:tt * vv + na, :]
        wtr_ref[:] = w_ref[tt * vv + na:, :]

    dn = (((0,), (0,)), ((), ()))
    acc = lax.dot_general(x_ref[0, 0, :, 0, :], wx_ref[0], dn,
                          preferred_element_type=jnp.float32)
    for t in range(1, tt):
        acc += lax.dot_general(x_ref[0, t, :, 0, :], wx_ref[t], dn,
                               preferred_element_type=jnp.float32)
    acc += lax.dot_general(na_ref[:], wna_ref[:], dn,
                           preferred_element_type=jnp.float32)
    acc += lax.dot_general(tr_ref[:], wtr_ref[:], dn,
                           preferred_element_type=jnp.float32)
    out_ref[:] = acc + b_ref[:]


def kernel(x, node_attr_data, trainable_data, node_attr_hidden,
           trainable_hidden, edge_attr, W_data, b_data, W_hidden, b_hidden,
           W_edge, Wq, Wk, Wv, Wo, W_mlp1, W_mlp2, ln1_g, ln1_b, ln2_g,
           ln2_b, edge_index):
    B, Tt, Ens, G, Vv = x.shape
    NA = node_attr_data.shape[1]
    TR = trainable_data.shape[1]
    D = W_data.shape[1]

    xt = jnp.transpose(x, (0, 1, 4, 2, 3))
    nat = node_attr_data.T
    trt = trainable_data.T
    b2 = b_data.reshape(1, D)

    Bg = 10240
    grid = (pl.cdiv(G, Bg),)

    return pl.pallas_call(
        _embed_kernel,
        grid=grid,
        in_specs=[
            pl.BlockSpec((1, Tt, Vv, 1, Bg), lambda i: (0, 0, 0, 0, i)),
            pl.BlockSpec((NA, Bg), lambda i: (0, i)),
            pl.BlockSpec((TR, Bg), lambda i: (0, i)),
            pl.BlockSpec((Tt * Vv + NA + TR, D), lambda i: (0, 0)),
            pl.BlockSpec((1, D), lambda i: (0, 0)),
        ],
        out_specs=pl.BlockSpec((Bg, D), lambda i: (i, 0)),
        out_shape=jax.ShapeDtypeStruct((G, D), jnp.float32),
        scratch_shapes=[
            pltpu.VMEM((Tt, Vv, D), jnp.float32),
            pltpu.VMEM((NA, D), jnp.float32),
            pltpu.VMEM((TR, D), jnp.float32),
        ],
    )(xt, nat, trt, W_data, b2)


# speedup vs baseline: 1.0423x; 1.0067x over previous
import jax
import jax.numpy as jnp
from jax import lax
from jax.experimental import pallas as pl
from jax.experimental.pallas import tpu as pltpu


def _embed_kernel(x_ref, na_ref, tr_ref, w_ref, b_ref, out_ref,
                  wx_ref, wna_ref, wtr_ref):
    tt = x_ref.shape[1]
    vv = x_ref.shape[2]
    na = wna_ref.shape[0]

    @pl.when(pl.program_id(0) == 0)
    def _load_w_slabs():
        for t in range(tt):
            wx_ref[t] = w_ref[t * vv:(t + 1) * vv, :]
        wna_ref[:] = w_ref[tt * vv:tt * vv + na, :]
        wtr_ref[:] = w_ref[tt * vv + na:, :]

    dn = (((0,), (0,)), ((), ()))
    acc = lax.dot_general(x_ref[0, 0, :, 0, :], wx_ref[0], dn,
                          preferred_element_type=jnp.float32)
    for t in range(1, tt):
        acc += lax.dot_general(x_ref[0, t, :, 0, :], wx_ref[t], dn,
                               preferred_element_type=jnp.float32)
    acc += lax.dot_general(na_ref[:], wna_ref[:], dn,
                           preferred_element_type=jnp.float32)
    acc += lax.dot_general(tr_ref[:], wtr_ref[:], dn,
                           preferred_element_type=jnp.float32)
    out_ref[:] = acc + b_ref[:]


def kernel(x, node_attr_data, trainable_data, node_attr_hidden,
           trainable_hidden, edge_attr, W_data, b_data, W_hidden, b_hidden,
           W_edge, Wq, Wk, Wv, Wo, W_mlp1, W_mlp2, ln1_g, ln1_b, ln2_g,
           ln2_b, edge_index):
    B, Tt, Ens, G, Vv = x.shape
    NA = node_attr_data.shape[1]
    TR = trainable_data.shape[1]
    D = W_data.shape[1]

    xt = jnp.transpose(x, (0, 1, 4, 2, 3))
    nat = node_attr_data.T
    trt = trainable_data.T
    b2 = b_data.reshape(1, D)

    Bg = 10112
    grid = (pl.cdiv(G, Bg),)

    return pl.pallas_call(
        _embed_kernel,
        grid=grid,
        in_specs=[
            pl.BlockSpec((1, Tt, Vv, 1, Bg), lambda i: (0, 0, 0, 0, i)),
            pl.BlockSpec((NA, Bg), lambda i: (0, i)),
            pl.BlockSpec((TR, Bg), lambda i: (0, i)),
            pl.BlockSpec((Tt * Vv + NA + TR, D), lambda i: (0, 0)),
            pl.BlockSpec((1, D), lambda i: (0, 0)),
        ],
        out_specs=pl.BlockSpec((Bg, D), lambda i: (i, 0)),
        out_shape=jax.ShapeDtypeStruct((G, D), jnp.float32),
        scratch_shapes=[
            pltpu.VMEM((Tt, Vv, D), jnp.float32),
            pltpu.VMEM((NA, D), jnp.float32),
            pltpu.VMEM((TR, D), jnp.float32),
        ],
    )(xt, nat, trt, W_data, b2)
